# baseline (speedup 1.0000x reference)
import functools
import math

import jax
import jax.numpy as jnp
from jax import lax
from jax.experimental import pallas as pl
from jax.experimental.pallas import tpu as pltpu
from jax.experimental.pallas import tpu_sc as plsc

_LOG_SLOPE = math.log(0.001)
_N = 256
_ROWS = 16384
_COLS = 512
_LANES = 16
_NWORKERS = 32
_RPW = _ROWS // _NWORKERS
_CH = 128
_NCHUNK = _RPW // _CH


def _sc_body(u_hbm, s_hbm, t_hbm, x_hbm, d_hbm,
             buf0, buf1, sv, tv, dv, sem_pass, sem0, sem1, sem_d):
    wid = lax.axis_index("s") * 2 + lax.axis_index("c")
    base = wid * _RPW

    pass_copy = pltpu.make_async_copy(
        u_hbm.at[pl.ds(base, _RPW), pl.ds(_N, _COLS - _N)],
        x_hbm.at[pl.ds(base, _RPW), pl.ds(_N, _COLS - _N)],
        sem_pass,
    )
    pass_copy.start()

    pltpu.sync_copy(s_hbm, sv)
    pltpu.sync_copy(t_hbm, tv)

    scales = []
    ts = []
    acc = jnp.zeros((_LANES,), jnp.float32)
    for k in range(_N // _LANES):
        s = sv[pl.ds(k * _LANES, _LANES)]
        logs = s / (1.0 + jnp.abs(s * (1.0 / _LOG_SLOPE)))
        scales.append(jnp.exp(logs))
        ts.append(tv[pl.ds(k * _LANES, _LANES)])
        acc = acc + logs
    total = acc[0]
    for i in range(1, _LANES):
        total = total + acc[i]

    dvec = jnp.full((_LANES,), total, dtype=jnp.float32)
    for i in range(_RPW // _LANES):
        dv[pl.ds(i * _LANES, _LANES)] = dvec
    d_copy = pltpu.make_async_copy(dv, d_hbm.at[pl.ds(base, _RPW)], sem_d)
    d_copy.start()

    def _affine(buf):
        def row_body(r, _):
            for k in range(_N // _LANES):
                sl = pl.ds(k * _LANES, _LANES)
                buf[r, sl] = buf[r, sl] * scales[k] + ts[k]
            return 0
        lax.fori_loop(0, _CH, row_body, 0)

    bufs = (buf0, buf1)
    pltpu.make_async_copy(
        u_hbm.at[pl.ds(base, _CH), pl.ds(0, _N)], buf0, sem0).start()
    for c in range(_NCHUNK):
        buf = bufs[c % 2]
        nxt = bufs[(c + 1) % 2]
        pltpu.make_async_copy(
            u_hbm.at[pl.ds(base + c * _CH, _CH), pl.ds(0, _N)], buf,
            sem0).wait()
        if c + 1 < _NCHUNK:
            if c >= 1:
                pltpu.make_async_copy(
                    nxt, x_hbm.at[pl.ds(base + (c - 1) * _CH, _CH),
                                  pl.ds(0, _N)], sem1).wait()
            pltpu.make_async_copy(
                u_hbm.at[pl.ds(base + (c + 1) * _CH, _CH), pl.ds(0, _N)],
                nxt, sem0).start()
        _affine(buf)
        pltpu.make_async_copy(
            buf, x_hbm.at[pl.ds(base + c * _CH, _CH), pl.ds(0, _N)],
            sem1).start()
    pltpu.make_async_copy(
        bufs[(_NCHUNK - 2) % 2],
        x_hbm.at[pl.ds(base + (_NCHUNK - 2) * _CH, _CH), pl.ds(0, _N)],
        sem1).wait()
    pltpu.make_async_copy(
        bufs[(_NCHUNK - 1) % 2],
        x_hbm.at[pl.ds(base + (_NCHUNK - 1) * _CH, _CH), pl.ds(0, _N)],
        sem1).wait()
    d_copy.wait()
    pass_copy.wait()


_sc_kernel = functools.partial(
    pl.kernel,
    out_type=[
        jax.ShapeDtypeStruct((_ROWS, _COLS), jnp.float32),
        jax.ShapeDtypeStruct((_ROWS,), jnp.float32),
    ],
    mesh=plsc.VectorSubcoreMesh(core_axis_name="c", subcore_axis_name="s"),
    scratch_types=[
        pltpu.VMEM((_CH, _N), jnp.float32),
        pltpu.VMEM((_CH, _N), jnp.float32),
        pltpu.VMEM((_N,), jnp.float32),
        pltpu.VMEM((_N,), jnp.float32),
        pltpu.VMEM((_RPW,), jnp.float32),
        pltpu.SemaphoreType.DMA,
        pltpu.SemaphoreType.DMA,
        pltpu.SemaphoreType.DMA,
        pltpu.SemaphoreType.DMA,
    ],
)(_sc_body)


@jax.jit
def kernel(u, un_s, t):
    return tuple(_sc_kernel(u, un_s, t))


# speedup vs baseline: 11.1245x; 11.1245x over previous
import functools
import math

import jax
import jax.numpy as jnp
from jax import lax
from jax.experimental import pallas as pl
from jax.experimental.pallas import tpu as pltpu
from jax.experimental.pallas import tpu_sc as plsc

_LOG_SLOPE = math.log(0.001)
_N = 256
_ROWS = 16384
_COLS = 512
_LANES = 16
_NWORKERS = 32
_RPW = _ROWS // _NWORKERS
_CH = 64
_NCHUNK = _RPW // _CH


def _sc_body(u_hbm, s_hbm, t_hbm, x_hbm, d_hbm,
             buf0, buf1, sv, tv, dv, sem_in, sem_out, sem_d):
    wid = lax.axis_index("s") * 2 + lax.axis_index("c")
    base = wid * _RPW

    pltpu.sync_copy(s_hbm, sv)
    pltpu.sync_copy(t_hbm, tv)

    scales = []
    ts = []
    acc = jnp.zeros((_LANES,), jnp.float32)
    for k in range(_N // _LANES):
        s = sv[pl.ds(k * _LANES, _LANES)]
        logs = s / (1.0 + jnp.abs(s * (1.0 / _LOG_SLOPE)))
        scales.append(jnp.exp(logs))
        ts.append(tv[pl.ds(k * _LANES, _LANES)])
        acc = acc + logs
    total = acc[0]
    for i in range(1, _LANES):
        total = total + acc[i]

    dvec = jnp.full((_LANES,), total, dtype=jnp.float32)
    for i in range(_RPW // _LANES):
        dv[pl.ds(i * _LANES, _LANES)] = dvec
    d_copy = pltpu.make_async_copy(dv, d_hbm.at[pl.ds(base, _RPW)], sem_d)
    d_copy.start()

    def _in_copy(c, buf):
        return pltpu.make_async_copy(
            u_hbm.at[pl.ds(base + c * _CH, _CH), :], buf, sem_in)

    def _out_copy(c, buf):
        return pltpu.make_async_copy(
            buf, x_hbm.at[pl.ds(base + c * _CH, _CH), :], sem_out)

    def _affine(buf):
        def row_body(r, _):
            for k in range(_N // _LANES):
                sl = pl.ds(k * _LANES, _LANES)
                buf[r, sl] = buf[r, sl] * scales[k] + ts[k]
            return 0
        lax.fori_loop(0, _CH, row_body, 0)

    bufs = (buf0, buf1)
    _in_copy(0, buf0).start()
    for c in range(_NCHUNK):
        buf = bufs[c % 2]
        nxt = bufs[(c + 1) % 2]
        _in_copy(c, buf).wait()
        if c + 1 < _NCHUNK:
            if c >= 1:
                _out_copy(c - 1, nxt).wait()
            _in_copy(c + 1, nxt).start()
        _affine(buf)
        _out_copy(c, buf).start()
    _out_copy(_NCHUNK - 2, bufs[(_NCHUNK - 2) % 2]).wait()
    _out_copy(_NCHUNK - 1, bufs[(_NCHUNK - 1) % 2]).wait()
    d_copy.wait()


_sc_kernel = functools.partial(
    pl.kernel,
    out_type=[
        jax.ShapeDtypeStruct((_ROWS, _COLS), jnp.float32),
        jax.ShapeDtypeStruct((_ROWS,), jnp.float32),
    ],
    mesh=plsc.VectorSubcoreMesh(core_axis_name="c", subcore_axis_name="s"),
    scratch_types=[
        pltpu.VMEM((_CH, _COLS), jnp.float32),
        pltpu.VMEM((_CH, _COLS), jnp.float32),
        pltpu.VMEM((_N,), jnp.float32),
        pltpu.VMEM((_N,), jnp.float32),
        pltpu.VMEM((_RPW,), jnp.float32),
        pltpu.SemaphoreType.DMA,
        pltpu.SemaphoreType.DMA,
        pltpu.SemaphoreType.DMA,
    ],
)(_sc_body)


@jax.jit
def kernel(u, un_s, t):
    return tuple(_sc_kernel(u, un_s, t))


# speedup vs baseline: 13.5534x; 1.2183x over previous
import functools
import math

import jax
import jax.numpy as jnp
from jax import lax
from jax.experimental import pallas as pl
from jax.experimental.pallas import tpu as pltpu
from jax.experimental.pallas import tpu_sc as plsc

_LOG_SLOPE = math.log(0.001)
_N = 256
_ROWS = 16384
_COLS = 512
_LANES = 16
_NWORKERS = 32
_RPW = _ROWS // _NWORKERS
_BLK_ROWS = 4096


def _logd_sc_body(s_hbm, d_hbm, sv, dv, sem_d):
    wid = lax.axis_index("s") * 2 + lax.axis_index("c")
    base = wid * _RPW

    pltpu.sync_copy(s_hbm, sv)
    acc = jnp.zeros((_LANES,), jnp.float32)
    for k in range(_N // _LANES):
        s = sv[pl.ds(k * _LANES, _LANES)]
        acc = acc + s / (1.0 + jnp.abs(s * (1.0 / _LOG_SLOPE)))
    total = acc[0]
    for i in range(1, _LANES):
        total = total + acc[i]

    dvec = jnp.full((_LANES,), total, dtype=jnp.float32)
    for i in range(_RPW // _LANES):
        dv[pl.ds(i * _LANES, _LANES)] = dvec
    pltpu.make_async_copy(dv, d_hbm.at[pl.ds(base, _RPW)], sem_d).start()
    pltpu.make_async_copy(dv, d_hbm.at[pl.ds(base, _RPW)], sem_d).wait()


_logd_sc = functools.partial(
    pl.kernel,
    out_type=jax.ShapeDtypeStruct((_ROWS,), jnp.float32),
    mesh=plsc.VectorSubcoreMesh(core_axis_name="c", subcore_axis_name="s"),
    scratch_types=[
        pltpu.VMEM((_N,), jnp.float32),
        pltpu.VMEM((_RPW,), jnp.float32),
        pltpu.SemaphoreType.DMA,
    ],
)(_logd_sc_body)


def _affine_tc_body(u_ref, s_ref, t_ref, x_ref):
    s = s_ref[0, :]
    logs = s / (1.0 + jnp.abs(s * (1.0 / _LOG_SLOPE)))
    scale = jnp.exp(logs)
    x_ref[:, :_N] = u_ref[:, :_N] * scale[None, :] + t_ref[0, :][None, :]
    x_ref[:, _N:] = u_ref[:, _N:]


@jax.jit
def kernel(u, un_s, t):
    logd = _logd_sc(un_s)
    x = pl.pallas_call(
        _affine_tc_body,
        grid=(_ROWS // _BLK_ROWS,),
        in_specs=[
            pl.BlockSpec((_BLK_ROWS, _COLS), lambda i: (i, 0)),
            pl.BlockSpec((1, _N), lambda i: (0, 0)),
            pl.BlockSpec((1, _N), lambda i: (0, 0)),
        ],
        out_specs=pl.BlockSpec((_BLK_ROWS, _COLS), lambda i: (i, 0)),
        out_shape=jax.ShapeDtypeStruct((_ROWS, _COLS), jnp.float32),
    )(u, un_s.reshape(1, _N), t.reshape(1, _N))
    return (x, logd)


# speedup vs baseline: 14.6453x; 1.0806x over previous
import functools
import math

import jax
import jax.numpy as jnp
from jax import lax
from jax.experimental import pallas as pl
from jax.experimental.pallas import tpu as pltpu
from jax.experimental.pallas import tpu_sc as plsc

_LOG_SLOPE = math.log(0.001)
_N = 256
_ROWS = 16384
_COLS = 512
_LANES = 16
_NWORKERS = 16
_RPW = _ROWS // _NWORKERS
_BLK_ROWS = 4096


def _logd_sc_body(s_hbm, d_hbm, sv, dv, sem_d):
    wid = lax.axis_index("s")
    base = wid * _RPW

    pltpu.sync_copy(s_hbm, sv)
    acc = jnp.zeros((_LANES,), jnp.float32)
    for k in range(_N // _LANES):
        s = sv[pl.ds(k * _LANES, _LANES)]
        acc = acc + s / (1.0 + jnp.abs(s * (1.0 / _LOG_SLOPE)))
    total = acc[0]
    for i in range(1, _LANES):
        total = total + acc[i]

    dvec = jnp.full((_LANES,), total, dtype=jnp.float32)
    for i in range(_RPW // _LANES):
        dv[pl.ds(i * _LANES, _LANES)] = dvec
    pltpu.make_async_copy(dv, d_hbm.at[pl.ds(base, _RPW)], sem_d).start()
    pltpu.make_async_copy(dv, d_hbm.at[pl.ds(base, _RPW)], sem_d).wait()


_logd_sc = functools.partial(
    pl.kernel,
    out_type=jax.ShapeDtypeStruct((_ROWS,), jnp.float32),
    mesh=plsc.VectorSubcoreMesh(core_axis_name="c", subcore_axis_name="s",
                                num_cores=1),
    scratch_types=[
        pltpu.VMEM((_N,), jnp.float32),
        pltpu.VMEM((_RPW,), jnp.float32),
        pltpu.SemaphoreType.DMA,
    ],
)(_logd_sc_body)


def _affine_tc_body(u_ref, s_ref, t_ref, x_ref):
    s = s_ref[0, :]
    logs = s / (1.0 + jnp.abs(s * (1.0 / _LOG_SLOPE)))
    scale = jnp.exp(logs)
    x_ref[:, :_N] = u_ref[:, :_N] * scale[None, :] + t_ref[0, :][None, :]
    x_ref[:, _N:] = u_ref[:, _N:]


@jax.jit
def kernel(u, un_s, t):
    logd = _logd_sc(un_s)
    x = pl.pallas_call(
        _affine_tc_body,
        grid=(_ROWS // _BLK_ROWS,),
        in_specs=[
            pl.BlockSpec((_BLK_ROWS, _COLS), lambda i: (i, 0)),
            pl.BlockSpec((1, _N), lambda i: (0, 0)),
            pl.BlockSpec((1, _N), lambda i: (0, 0)),
        ],
        out_specs=pl.BlockSpec((_BLK_ROWS, _COLS), lambda i: (i, 0)),
        out_shape=jax.ShapeDtypeStruct((_ROWS, _COLS), jnp.float32),
    )(u, un_s.reshape(1, _N), t.reshape(1, _N))
    return (x, logd)


# speedup vs baseline: 14.7293x; 1.0057x over previous
import functools
import math

import jax
import jax.numpy as jnp
from jax import lax
from jax.experimental import pallas as pl
from jax.experimental.pallas import tpu as pltpu
from jax.experimental.pallas import tpu_sc as plsc

_LOG_SLOPE = math.log(0.001)
_N = 256
_ROWS = 16384
_COLS = 512
_LANES = 16
_NWORKERS = 16
_RPW = _ROWS // _NWORKERS
_BLK_ROWS = 4096


def _logd_sc_body(s_hbm, d_hbm, sv, dv, sem_d):
    wid = lax.axis_index("s")
    base = wid * _RPW

    pltpu.sync_copy(s_hbm, sv)
    acc = jnp.zeros((_LANES,), jnp.float32)
    for k in range(_N // _LANES):
        s = sv[pl.ds(k * _LANES, _LANES)]
        acc = acc + s / (1.0 + jnp.abs(s * (1.0 / _LOG_SLOPE)))
    total = acc[0]
    for i in range(1, _LANES):
        total = total + acc[i]

    dvec = jnp.full((_LANES,), total, dtype=jnp.float32)
    for i in range(_RPW // _LANES):
        dv[pl.ds(i * _LANES, _LANES)] = dvec
    pltpu.make_async_copy(dv, d_hbm.at[pl.ds(base, _RPW)], sem_d).start()
    pltpu.make_async_copy(dv, d_hbm.at[pl.ds(base, _RPW)], sem_d).wait()


_logd_sc = functools.partial(
    pl.kernel,
    out_type=jax.ShapeDtypeStruct((_ROWS,), jnp.float32),
    mesh=plsc.VectorSubcoreMesh(core_axis_name="c", subcore_axis_name="s",
                                num_cores=1),
    scratch_types=[
        pltpu.VMEM((_N,), jnp.float32),
        pltpu.VMEM((_RPW,), jnp.float32),
        pltpu.SemaphoreType.DMA,
    ],
)(_logd_sc_body)


def _affine_tc_body(u_ref, s_ref, t_ref, x_ref):
    s = s_ref[0, :]
    logs = s / (1.0 + jnp.abs(s * (1.0 / _LOG_SLOPE)))
    scale = jnp.exp(logs)
    x_ref[:, :_N] = u_ref[:, :_N] * scale[None, :] + t_ref[0, :][None, :]
    x_ref[:, _N:] = u_ref[:, _N:]


@jax.jit
def kernel(u, un_s, t):
    logd = _logd_sc(un_s + 0.0)
    x = pl.pallas_call(
        _affine_tc_body,
        grid=(_ROWS // _BLK_ROWS,),
        in_specs=[
            pl.BlockSpec((_BLK_ROWS, _COLS), lambda i: (i, 0)),
            pl.BlockSpec((1, _N), lambda i: (0, 0)),
            pl.BlockSpec((1, _N), lambda i: (0, 0)),
        ],
        out_specs=pl.BlockSpec((_BLK_ROWS, _COLS), lambda i: (i, 0)),
        out_shape=jax.ShapeDtypeStruct((_ROWS, _COLS), jnp.float32),
    )(u, un_s.reshape(1, _N), t.reshape(1, _N))
    return (x, logd)
